# baseline (speedup 1.0000x reference)
import functools

import jax
import jax.numpy as jnp
from jax import lax
from jax.experimental import pallas as pl
from jax.experimental.pallas import tpu as pltpu
from jax.experimental.pallas import tpu_sc as plsc

B = 16384
D = 128
NC = 2
NS = 16
NW = NC * NS
BPW = B // NW
CHUNK = 128
NCHUNK = BPW // CHUNK
NSLOT = 3
AHEAD = 2


def _body(u_hbm, i_hbm, emb_u_hbm, emb_i_hbm, out_hbm,
          uidx, iidx, ubuf, ibuf, out_v, *sems):
    cid = lax.axis_index("c")
    sid = lax.axis_index("s")
    wid = sid * NC + cid
    base = wid * BPW

    def start(j):
        slot = j % NSLOT
        pltpu.sync_copy(u_hbm.at[pl.ds(base + j * CHUNK, CHUNK)], uidx.at[j])
        pltpu.sync_copy(i_hbm.at[pl.ds(base + j * CHUNK, CHUNK)], iidx.at[j])
        cu = pltpu.async_copy(emb_u_hbm.at[uidx.at[j]], ubuf.at[slot],
                              sems[2 * slot])
        ci = pltpu.async_copy(emb_i_hbm.at[iidx.at[j]], ibuf.at[slot],
                              sems[2 * slot + 1])
        return cu, ci

    pending = [start(j) for j in range(AHEAD)]
    for j in range(NCHUNK):
        slot = j % NSLOT
        cu, ci = pending[j]
        cu.wait()
        ci.wait()
        if j + AHEAD < NCHUNK:
            pending.append(start(j + AHEAD))

        def block(b, carry, j=j, slot=slot):
            rows = lax.iota(jnp.int32, 16) + b * 16
            acc = jnp.zeros((16,), jnp.float32)
            for d in range(D):
                col = jnp.full((16,), d, jnp.int32)
                ug = plsc.load_gather(ubuf.at[slot], [rows, col])
                ig = plsc.load_gather(ibuf.at[slot], [rows, col])
                acc = acc + ug * ig
            out_v[pl.ds(j * CHUNK + b * 16, 16)] = acc
            return carry

        lax.fori_loop(0, CHUNK // 16, block, 0)

    pltpu.sync_copy(out_v, out_hbm.at[pl.ds(base, BPW)])


_sc_call = pl.kernel(
    _body,
    out_type=jax.ShapeDtypeStruct((B,), jnp.float32),
    mesh=plsc.VectorSubcoreMesh(
        core_axis_name="c", subcore_axis_name="s",
        num_cores=NC, num_subcores=NS),
    scratch_types=[
        pltpu.VMEM((NCHUNK, CHUNK), jnp.int32),
        pltpu.VMEM((NCHUNK, CHUNK), jnp.int32),
        pltpu.VMEM((NSLOT, CHUNK, D), jnp.float32),
        pltpu.VMEM((NSLOT, CHUNK, D), jnp.float32),
        pltpu.VMEM((BPW,), jnp.float32),
    ] + [pltpu.SemaphoreType.DMA] * (2 * NSLOT),
    compiler_params=pltpu.CompilerParams(needs_layout_passes=False, skip_device_barrier=True),
)


@jax.jit
def kernel(u, i, emb_u, emb_i):
    return _sc_call(u.astype(jnp.int32), i.astype(jnp.int32), emb_u, emb_i)


# speedup vs baseline: 2.5058x; 2.5058x over previous
import functools

import jax
import jax.numpy as jnp
from jax import lax
from jax.experimental import pallas as pl
from jax.experimental.pallas import tpu as pltpu
from jax.experimental.pallas import tpu_sc as plsc

B = 16384
D = 128
NC = 2
NS = 16
NW = NC * NS
BPW = B // NW
CHUNK = 128
NCHUNK = BPW // CHUNK
NSLOT = 3
AHEAD = 2


def _body(u_hbm, i_hbm, emb_u_hbm, emb_i_hbm, out_hbm,
          uidx, iidx, ubuf, ibuf, out_v, *sems):
    cid = lax.axis_index("c")
    sid = lax.axis_index("s")
    wid = sid * NC + cid
    base = wid * BPW

    pltpu.sync_copy(u_hbm.at[pl.ds(base, BPW)], uidx)
    pltpu.sync_copy(i_hbm.at[pl.ds(base, BPW)], iidx)

    def start(j):
        slot = j % NSLOT
        cu = pltpu.async_copy(
            emb_u_hbm.at[uidx.at[pl.ds(j * CHUNK, CHUNK)]], ubuf.at[slot],
            sems[2 * slot])
        ci = pltpu.async_copy(
            emb_i_hbm.at[iidx.at[pl.ds(j * CHUNK, CHUNK)]], ibuf.at[slot],
            sems[2 * slot + 1])
        return cu, ci

    pending = [start(j) for j in range(AHEAD)]
    for j in range(NCHUNK):
        slot = j % NSLOT
        cu, ci = pending[j]
        cu.wait()
        ci.wait()
        if j + AHEAD < NCHUNK:
            pending.append(start(j + AHEAD))

        def row(r, carry, j=j, slot=slot):
            acc = jnp.zeros((16,), jnp.float32)
            for d8 in range(D // 16):
                uv = ubuf[slot, r, pl.ds(d8 * 16, 16)]
                iv = ibuf[slot, r, pl.ds(d8 * 16, 16)]
                acc = acc + uv * iv
            tot = plsc.cumsum(acc)
            lane = lax.iota(jnp.int32, 16)
            pos = jnp.full((16,), j * CHUNK + r, jnp.int32)
            plsc.store_scatter(out_v, [pos], tot, mask=lane == 15)
            return carry

        lax.fori_loop(0, CHUNK, row, 0, unroll=8)

    pltpu.sync_copy(out_v, out_hbm.at[pl.ds(base, BPW)])


_sc_call = pl.kernel(
    _body,
    out_type=jax.ShapeDtypeStruct((B,), jnp.float32),
    mesh=plsc.VectorSubcoreMesh(
        core_axis_name="c", subcore_axis_name="s",
        num_cores=NC, num_subcores=NS),
    scratch_types=[
        pltpu.VMEM((BPW,), jnp.int32),
        pltpu.VMEM((BPW,), jnp.int32),
        pltpu.VMEM((NSLOT, CHUNK, D), jnp.float32),
        pltpu.VMEM((NSLOT, CHUNK, D), jnp.float32),
        pltpu.VMEM((BPW,), jnp.float32),
    ] + [pltpu.SemaphoreType.DMA] * (2 * NSLOT),
    compiler_params=pltpu.CompilerParams(needs_layout_passes=False,
                                         skip_device_barrier=True),
)


@jax.jit
def kernel(u, i, emb_u, emb_i):
    return _sc_call(u.astype(jnp.int32), i.astype(jnp.int32), emb_u, emb_i)


# speedup vs baseline: 2.6600x; 1.0615x over previous
import functools

import jax
import jax.numpy as jnp
from jax import lax
from jax.experimental import pallas as pl
from jax.experimental.pallas import tpu as pltpu
from jax.experimental.pallas import tpu_sc as plsc

B = 16384
D = 128
NC = 2
NS = 16
NW = NC * NS
BPW = B // NW
CHUNK = 128
NCHUNK = BPW // CHUNK
NSLOT = 3
AHEAD = 2


def _body(u_hbm, i_hbm, emb_u_hbm, emb_i_hbm, out_hbm,
          uidx, iidx, ubuf, ibuf, out_v, *sems):
    cid = lax.axis_index("c")
    sid = lax.axis_index("s")
    wid = sid * NC + cid
    base = wid * BPW

    pltpu.sync_copy(u_hbm.at[pl.ds(base, BPW)], uidx)
    pltpu.sync_copy(i_hbm.at[pl.ds(base, BPW)], iidx)

    def start(j):
        slot = j % NSLOT
        cu = pltpu.async_copy(
            emb_u_hbm.at[uidx.at[pl.ds(j * CHUNK, CHUNK)]], ubuf.at[slot],
            sems[2 * slot])
        ci = pltpu.async_copy(
            emb_i_hbm.at[iidx.at[pl.ds(j * CHUNK, CHUNK)]], ibuf.at[slot],
            sems[2 * slot + 1])
        return cu, ci

    pending = [start(j) for j in range(AHEAD)]
    for j in range(NCHUNK):
        slot = j % NSLOT
        cu, ci = pending[j]
        cu.wait()
        ci.wait()
        if j + AHEAD < NCHUNK:
            pending.append(start(j + AHEAD))

        def row(r, carry, j=j, slot=slot):
            acc = jnp.zeros((16,), jnp.float32)
            for d8 in range(D // 16):
                uv = ubuf[slot, r, pl.ds(d8 * 16, 16)]
                iv = ibuf[slot, r, pl.ds(d8 * 16, 16)]
                acc = acc + uv * iv
            tot = plsc.cumsum(acc)
            lane = lax.iota(jnp.int32, 16)
            pos = jnp.full((16,), j * CHUNK + r, jnp.int32)
            plsc.store_scatter(out_v, [pos], tot, mask=lane == 15)
            return carry

        lax.fori_loop(0, CHUNK, row, 0, unroll=4)

    pltpu.sync_copy(out_v, out_hbm.at[pl.ds(base, BPW)])


_sc_call = pl.kernel(
    _body,
    out_type=jax.ShapeDtypeStruct((B,), jnp.float32),
    mesh=plsc.VectorSubcoreMesh(
        core_axis_name="c", subcore_axis_name="s",
        num_cores=NC, num_subcores=NS),
    scratch_types=[
        pltpu.VMEM((BPW,), jnp.int32),
        pltpu.VMEM((BPW,), jnp.int32),
        pltpu.VMEM((NSLOT, CHUNK, D), jnp.float32),
        pltpu.VMEM((NSLOT, CHUNK, D), jnp.float32),
        pltpu.VMEM((BPW,), jnp.float32),
    ] + [pltpu.SemaphoreType.DMA] * (2 * NSLOT),
    compiler_params=pltpu.CompilerParams(needs_layout_passes=False,
                                         skip_device_barrier=True),
)


@jax.jit
def kernel(u, i, emb_u, emb_i):
    return _sc_call(u.astype(jnp.int32), i.astype(jnp.int32), emb_u, emb_i)


# speedup vs baseline: 2.7017x; 1.0156x over previous
import functools

import jax
import jax.numpy as jnp
from jax import lax
from jax.experimental import pallas as pl
from jax.experimental.pallas import tpu as pltpu
from jax.experimental.pallas import tpu_sc as plsc

B = 16384
D = 128
NC = 2
NS = 16
NW = NC * NS
BPW = B // NW
CHUNK = 128
NCHUNK = BPW // CHUNK
NSLOT = 3
AHEAD = 2


def _body(u_hbm, i_hbm, emb_u_hbm, emb_i_hbm, out_hbm,
          uidx, iidx, ubuf, ibuf, out_v, *sems):
    cid = lax.axis_index("c")
    sid = lax.axis_index("s")
    wid = sid * NC + cid
    base = wid * BPW

    pltpu.sync_copy(u_hbm.at[pl.ds(base, BPW)], uidx)
    pltpu.sync_copy(i_hbm.at[pl.ds(base, BPW)], iidx)

    def start(j):
        slot = j % NSLOT
        cu = pltpu.async_copy(
            emb_u_hbm.at[uidx.at[pl.ds(j * CHUNK, CHUNK)]], ubuf.at[slot],
            sems[2 * slot])
        ci = pltpu.async_copy(
            emb_i_hbm.at[iidx.at[pl.ds(j * CHUNK, CHUNK)]], ibuf.at[slot],
            sems[2 * slot + 1])
        return cu, ci

    pending = [start(j) for j in range(AHEAD)]
    for j in range(NCHUNK):
        slot = j % NSLOT
        cu, ci = pending[j]
        cu.wait()
        ci.wait()
        if j + AHEAD < NCHUNK:
            pending.append(start(j + AHEAD))

        def row(r, carry, j=j, slot=slot):
            acc = jnp.zeros((16,), jnp.float32)
            for d8 in range(D // 16):
                uv = ubuf[slot, r, pl.ds(d8 * 16, 16)]
                iv = ibuf[slot, r, pl.ds(d8 * 16, 16)]
                acc = acc + uv * iv
            tot = plsc.cumsum(acc)
            lane = lax.iota(jnp.int32, 16)
            pos = jnp.full((16,), j * CHUNK + r, jnp.int32)
            plsc.store_scatter(out_v, [pos], tot, mask=lane == 15)
            return carry

        lax.fori_loop(0, CHUNK, row, 0, unroll=2)

    pltpu.sync_copy(out_v, out_hbm.at[pl.ds(base, BPW)])


_sc_call = pl.kernel(
    _body,
    out_type=jax.ShapeDtypeStruct((B,), jnp.float32),
    mesh=plsc.VectorSubcoreMesh(
        core_axis_name="c", subcore_axis_name="s",
        num_cores=NC, num_subcores=NS),
    scratch_types=[
        pltpu.VMEM((BPW,), jnp.int32),
        pltpu.VMEM((BPW,), jnp.int32),
        pltpu.VMEM((NSLOT, CHUNK, D), jnp.float32),
        pltpu.VMEM((NSLOT, CHUNK, D), jnp.float32),
        pltpu.VMEM((BPW,), jnp.float32),
    ] + [pltpu.SemaphoreType.DMA] * (2 * NSLOT),
    compiler_params=pltpu.CompilerParams(needs_layout_passes=False,
                                         skip_device_barrier=True),
)


@jax.jit
def kernel(u, i, emb_u, emb_i):
    return _sc_call(u.astype(jnp.int32), i.astype(jnp.int32), emb_u, emb_i)


# speedup vs baseline: 2.7647x; 1.0234x over previous
import functools

import jax
import jax.numpy as jnp
from jax import lax
from jax.experimental import pallas as pl
from jax.experimental.pallas import tpu as pltpu
from jax.experimental.pallas import tpu_sc as plsc

B = 16384
D = 128
NC = 2
NS = 16
NW = NC * NS
BPW = B // NW
CHUNK = 128
NCHUNK = BPW // CHUNK
NSLOT = 3
AHEAD = 2


def _body(u_hbm, i_hbm, emb_u_hbm, emb_i_hbm, out_hbm,
          uidx, iidx, ubuf, ibuf, out_v, *sems):
    cid = lax.axis_index("c")
    sid = lax.axis_index("s")
    wid = sid * NC + cid
    base = wid * BPW

    pltpu.sync_copy(u_hbm.at[pl.ds(base, BPW)], uidx)
    pltpu.sync_copy(i_hbm.at[pl.ds(base, BPW)], iidx)

    def start(j):
        slot = j % NSLOT
        cu = pltpu.async_copy(
            emb_u_hbm.at[uidx.at[pl.ds(j * CHUNK, CHUNK)]], ubuf.at[slot],
            sems[2 * slot])
        ci = pltpu.async_copy(
            emb_i_hbm.at[iidx.at[pl.ds(j * CHUNK, CHUNK)]], ibuf.at[slot],
            sems[2 * slot + 1])
        return cu, ci

    pending = [start(j) for j in range(AHEAD)]
    for j in range(NCHUNK):
        slot = j % NSLOT
        cu, ci = pending[j]
        cu.wait()
        ci.wait()
        if j + AHEAD < NCHUNK:
            pending.append(start(j + AHEAD))

        def row(r, carry, j=j, slot=slot):
            acc = jnp.zeros((16,), jnp.float32)
            for d8 in range(D // 16):
                uv = ubuf[slot, r, pl.ds(d8 * 16, 16)]
                iv = ibuf[slot, r, pl.ds(d8 * 16, 16)]
                acc = acc + uv * iv
            tot = plsc.cumsum(acc)
            lane = lax.iota(jnp.int32, 16)
            pos = jnp.full((16,), j * CHUNK + r, jnp.int32)
            plsc.store_scatter(out_v, [pos], tot, mask=lane == 15)
            return carry

        lax.fori_loop(0, CHUNK, row, 0)

    pltpu.sync_copy(out_v, out_hbm.at[pl.ds(base, BPW)])


_sc_call = pl.kernel(
    _body,
    out_type=jax.ShapeDtypeStruct((B,), jnp.float32),
    mesh=plsc.VectorSubcoreMesh(
        core_axis_name="c", subcore_axis_name="s",
        num_cores=NC, num_subcores=NS),
    scratch_types=[
        pltpu.VMEM((BPW,), jnp.int32),
        pltpu.VMEM((BPW,), jnp.int32),
        pltpu.VMEM((NSLOT, CHUNK, D), jnp.float32),
        pltpu.VMEM((NSLOT, CHUNK, D), jnp.float32),
        pltpu.VMEM((BPW,), jnp.float32),
    ] + [pltpu.SemaphoreType.DMA] * (2 * NSLOT),
    compiler_params=pltpu.CompilerParams(needs_layout_passes=False,
                                         skip_device_barrier=True),
)


@jax.jit
def kernel(u, i, emb_u, emb_i):
    return _sc_call(u.astype(jnp.int32), i.astype(jnp.int32), emb_u, emb_i)


# speedup vs baseline: 2.7735x; 1.0032x over previous
import functools

import jax
import jax.numpy as jnp
from jax import lax
from jax.experimental import pallas as pl
from jax.experimental.pallas import tpu as pltpu
from jax.experimental.pallas import tpu_sc as plsc

B = 16384
D = 128
NC = 2
NS = 16
NW = NC * NS
BPW = B // NW
CHUNK = 128
NCHUNK = BPW // CHUNK
NSLOT = 3


def _body(u_hbm, i_hbm, emb_u_hbm, emb_i_hbm, out_hbm,
          uidx, iidx, ubuf, ibuf, out_v, semu, semi):
    cid = lax.axis_index("c")
    sid = lax.axis_index("s")
    wid = sid * NC + cid
    base = wid * BPW

    pltpu.sync_copy(u_hbm.at[pl.ds(base, BPW)], uidx)
    pltpu.sync_copy(i_hbm.at[pl.ds(base, BPW)], iidx)

    def start(j, slot):
        pltpu.async_copy(
            emb_u_hbm.at[uidx.at[pl.ds(j * CHUNK, CHUNK)]], ubuf.at[slot],
            semu.at[slot])
        pltpu.async_copy(
            emb_i_hbm.at[iidx.at[pl.ds(j * CHUNK, CHUNK)]], ibuf.at[slot],
            semi.at[slot])

    def wait(j, slot):
        pltpu.make_async_copy(
            emb_u_hbm.at[uidx.at[pl.ds(j * CHUNK, CHUNK)]], ubuf.at[slot],
            semu.at[slot]).wait()
        pltpu.make_async_copy(
            emb_i_hbm.at[iidx.at[pl.ds(j * CHUNK, CHUNK)]], ibuf.at[slot],
            semi.at[slot]).wait()

    start(0, 0)
    start(1, 1)

    def chunk(j, carry):
        slot = lax.rem(j, NSLOT)
        wait(j, slot)

        @pl.when(j + 2 < NCHUNK)
        def _():
            start(j + 2, lax.rem(j + 2, NSLOT))

        def row(r, carry2):
            acc = jnp.zeros((16,), jnp.float32)
            for d8 in range(D // 16):
                uv = ubuf[slot, r, pl.ds(d8 * 16, 16)]
                iv = ibuf[slot, r, pl.ds(d8 * 16, 16)]
                acc = acc + uv * iv
            tot = plsc.cumsum(acc)
            lane = lax.iota(jnp.int32, 16)
            pos = jnp.full((16,), j * CHUNK + r, jnp.int32)
            plsc.store_scatter(out_v, [pos], tot, mask=lane == 15)
            return carry2

        lax.fori_loop(0, CHUNK, row, 0)
        return carry

    lax.fori_loop(0, NCHUNK, chunk, 0)
    pltpu.sync_copy(out_v, out_hbm.at[pl.ds(base, BPW)])


_sc_call = pl.kernel(
    _body,
    out_type=jax.ShapeDtypeStruct((B,), jnp.float32),
    mesh=plsc.VectorSubcoreMesh(
        core_axis_name="c", subcore_axis_name="s",
        num_cores=NC, num_subcores=NS),
    scratch_types=[
        pltpu.VMEM((BPW,), jnp.int32),
        pltpu.VMEM((BPW,), jnp.int32),
        pltpu.VMEM((NSLOT, CHUNK, D), jnp.float32),
        pltpu.VMEM((NSLOT, CHUNK, D), jnp.float32),
        pltpu.VMEM((BPW,), jnp.float32),
        pltpu.SemaphoreType.DMA((NSLOT,)),
        pltpu.SemaphoreType.DMA((NSLOT,)),
    ],
    compiler_params=pltpu.CompilerParams(needs_layout_passes=False,
                                         skip_device_barrier=True),
)


@jax.jit
def kernel(u, i, emb_u, emb_i):
    return _sc_call(u.astype(jnp.int32), i.astype(jnp.int32), emb_u, emb_i)


# speedup vs baseline: 2.8053x; 1.0115x over previous
import jax
import jax.numpy as jnp
from jax import lax
from jax.experimental import pallas as pl
from jax.experimental.pallas import tpu as pltpu
from jax.experimental.pallas import tpu_sc as plsc

B = 16384
D = 128
NC = 2
NS = 16
NW = NC * NS
BPW = B // NW
CHUNK = 128
TAIL = 64
NFULL = 3
NVIRT = 5
NSLOT = 3


def _body(u_hbm, i_hbm, emb_u_hbm, emb_i_hbm, out_hbm,
          uidx, iidx, ubuf, ibuf, out_v, semu, semi):
    cid = lax.axis_index("c")
    sid = lax.axis_index("s")
    wid = sid * NC + cid
    base = wid * BPW

    pltpu.sync_copy(u_hbm.at[pl.ds(base, BPW)], uidx)
    pltpu.sync_copy(i_hbm.at[pl.ds(base, BPW)], iidx)

    def off_of(j):
        return j * CHUNK - jnp.maximum(j - NFULL, 0) * TAIL

    def start(j, slot):
        off = off_of(j)

        @pl.when(j < NFULL)
        def _():
            pltpu.async_copy(emb_u_hbm.at[uidx.at[pl.ds(off, CHUNK)]],
                             ubuf.at[slot].at[pl.ds(0, CHUNK)], semu.at[slot])
            pltpu.async_copy(emb_i_hbm.at[iidx.at[pl.ds(off, CHUNK)]],
                             ibuf.at[slot].at[pl.ds(0, CHUNK)], semi.at[slot])

        @pl.when(j >= NFULL)
        def _():
            pltpu.async_copy(emb_u_hbm.at[uidx.at[pl.ds(off, TAIL)]],
                             ubuf.at[slot].at[pl.ds(0, TAIL)], semu.at[slot])
            pltpu.async_copy(emb_i_hbm.at[iidx.at[pl.ds(off, TAIL)]],
                             ibuf.at[slot].at[pl.ds(0, TAIL)], semi.at[slot])

    def wait(j, slot):
        @pl.when(j < NFULL)
        def _():
            pltpu.make_async_copy(
                emb_u_hbm.at[uidx.at[pl.ds(0, CHUNK)]],
                ubuf.at[slot].at[pl.ds(0, CHUNK)], semu.at[slot]).wait()
            pltpu.make_async_copy(
                emb_i_hbm.at[iidx.at[pl.ds(0, CHUNK)]],
                ibuf.at[slot].at[pl.ds(0, CHUNK)], semi.at[slot]).wait()

        @pl.when(j >= NFULL)
        def _():
            pltpu.make_async_copy(
                emb_u_hbm.at[uidx.at[pl.ds(0, TAIL)]],
                ubuf.at[slot].at[pl.ds(0, TAIL)], semu.at[slot]).wait()
            pltpu.make_async_copy(
                emb_i_hbm.at[iidx.at[pl.ds(0, TAIL)]],
                ibuf.at[slot].at[pl.ds(0, TAIL)], semi.at[slot]).wait()

    start(0, 0)
    start(1, 1)

    def chunk(j, carry):
        slot = lax.rem(j, NSLOT)
        wait(j, slot)

        @pl.when(j + 2 < NVIRT)
        def _():
            start(j + 2, lax.rem(j + 2, NSLOT))

        n = jnp.where(j < NFULL, CHUNK, TAIL)
        out_off = off_of(j)

        def row(r, carry2):
            acc = jnp.zeros((16,), jnp.float32)
            for d8 in range(D // 16):
                uv = ubuf[slot, r, pl.ds(d8 * 16, 16)]
                iv = ibuf[slot, r, pl.ds(d8 * 16, 16)]
                acc = acc + uv * iv
            tot = plsc.cumsum(acc)
            lane = lax.iota(jnp.int32, 16)
            pos = jnp.full((16,), out_off + r, jnp.int32)
            plsc.store_scatter(out_v, [pos], tot, mask=lane == 15)
            return carry2

        lax.fori_loop(0, n, row, 0)
        return carry

    lax.fori_loop(0, NVIRT, chunk, 0)
    pltpu.sync_copy(out_v, out_hbm.at[pl.ds(base, BPW)])


_sc_call = pl.kernel(
    _body,
    out_type=jax.ShapeDtypeStruct((B,), jnp.float32),
    mesh=plsc.VectorSubcoreMesh(
        core_axis_name="c", subcore_axis_name="s",
        num_cores=NC, num_subcores=NS),
    scratch_types=[
        pltpu.VMEM((BPW,), jnp.int32),
        pltpu.VMEM((BPW,), jnp.int32),
        pltpu.VMEM((NSLOT, CHUNK, D), jnp.float32),
        pltpu.VMEM((NSLOT, CHUNK, D), jnp.float32),
        pltpu.VMEM((BPW,), jnp.float32),
        pltpu.SemaphoreType.DMA((NSLOT,)),
        pltpu.SemaphoreType.DMA((NSLOT,)),
    ],
    compiler_params=pltpu.CompilerParams(needs_layout_passes=False,
                                         skip_device_barrier=True),
)


@jax.jit
def kernel(u, i, emb_u, emb_i):
    return _sc_call(u.astype(jnp.int32), i.astype(jnp.int32), emb_u, emb_i)


# speedup vs baseline: 2.8310x; 1.0092x over previous
import jax
import jax.numpy as jnp
from jax import lax
from jax.experimental import pallas as pl
from jax.experimental.pallas import tpu as pltpu
from jax.experimental.pallas import tpu_sc as plsc

B = 16384
D = 128
NC = 2
NS = 16
NW = NC * NS
BPW = B // NW
CHUNK = 128
TAIL = 64
NFULL = 3
NVIRT = 5
NSLOT = 3


def _body(u_hbm, i_hbm, emb_u_hbm, emb_i_hbm, out_hbm,
          uidx, iidx, ubuf, ibuf, out_v, semu, semi):
    cid = lax.axis_index("c")
    sid = lax.axis_index("s")
    wid = sid * NC + cid
    base = wid * BPW

    cu0 = pltpu.async_copy(u_hbm.at[pl.ds(base, BPW)], uidx, semu.at[0])
    ci0 = pltpu.async_copy(i_hbm.at[pl.ds(base, BPW)], iidx, semi.at[0])
    cu0.wait()
    ci0.wait()

    def off_of(j):
        return j * CHUNK - jnp.maximum(j - NFULL, 0) * TAIL

    def start(j, slot):
        off = off_of(j)

        @pl.when(j < NFULL)
        def _():
            pltpu.async_copy(emb_u_hbm.at[uidx.at[pl.ds(off, CHUNK)]],
                             ubuf.at[slot].at[pl.ds(0, CHUNK)], semu.at[slot])
            pltpu.async_copy(emb_i_hbm.at[iidx.at[pl.ds(off, CHUNK)]],
                             ibuf.at[slot].at[pl.ds(0, CHUNK)], semi.at[slot])

        @pl.when(j >= NFULL)
        def _():
            pltpu.async_copy(emb_u_hbm.at[uidx.at[pl.ds(off, TAIL)]],
                             ubuf.at[slot].at[pl.ds(0, TAIL)], semu.at[slot])
            pltpu.async_copy(emb_i_hbm.at[iidx.at[pl.ds(off, TAIL)]],
                             ibuf.at[slot].at[pl.ds(0, TAIL)], semi.at[slot])

    def wait(j, slot):
        @pl.when(j < NFULL)
        def _():
            pltpu.make_async_copy(
                emb_u_hbm.at[uidx.at[pl.ds(0, CHUNK)]],
                ubuf.at[slot].at[pl.ds(0, CHUNK)], semu.at[slot]).wait()
            pltpu.make_async_copy(
                emb_i_hbm.at[iidx.at[pl.ds(0, CHUNK)]],
                ibuf.at[slot].at[pl.ds(0, CHUNK)], semi.at[slot]).wait()

        @pl.when(j >= NFULL)
        def _():
            pltpu.make_async_copy(
                emb_u_hbm.at[uidx.at[pl.ds(0, TAIL)]],
                ubuf.at[slot].at[pl.ds(0, TAIL)], semu.at[slot]).wait()
            pltpu.make_async_copy(
                emb_i_hbm.at[iidx.at[pl.ds(0, TAIL)]],
                ibuf.at[slot].at[pl.ds(0, TAIL)], semi.at[slot]).wait()

    start(0, 0)
    start(1, 1)

    def chunk(j, carry):
        slot = lax.rem(j, NSLOT)
        wait(j, slot)

        @pl.when(j + 2 < NVIRT)
        def _():
            start(j + 2, lax.rem(j + 2, NSLOT))

        n = jnp.where(j < NFULL, CHUNK, TAIL)
        out_off = off_of(j)

        def row(r, carry2):
            acc = jnp.zeros((16,), jnp.float32)
            for d8 in range(D // 16):
                uv = ubuf[slot, r, pl.ds(d8 * 16, 16)]
                iv = ibuf[slot, r, pl.ds(d8 * 16, 16)]
                acc = acc + uv * iv
            tot = plsc.cumsum(acc)
            lane = lax.iota(jnp.int32, 16)
            pos = jnp.full((16,), out_off + r, jnp.int32)
            plsc.store_scatter(out_v, [pos], tot, mask=lane == 15)
            return carry2

        lax.fori_loop(0, n, row, 0)
        return carry

    lax.fori_loop(0, NVIRT, chunk, 0)
    pltpu.sync_copy(out_v, out_hbm.at[pl.ds(base, BPW)])


_sc_call = pl.kernel(
    _body,
    out_type=jax.ShapeDtypeStruct((B,), jnp.float32),
    mesh=plsc.VectorSubcoreMesh(
        core_axis_name="c", subcore_axis_name="s",
        num_cores=NC, num_subcores=NS),
    scratch_types=[
        pltpu.VMEM((BPW,), jnp.int32),
        pltpu.VMEM((BPW,), jnp.int32),
        pltpu.VMEM((NSLOT, CHUNK, D), jnp.float32),
        pltpu.VMEM((NSLOT, CHUNK, D), jnp.float32),
        pltpu.VMEM((BPW,), jnp.float32),
        pltpu.SemaphoreType.DMA((NSLOT,)),
        pltpu.SemaphoreType.DMA((NSLOT,)),
    ],
    compiler_params=pltpu.CompilerParams(needs_layout_passes=False,
                                         skip_device_barrier=True),
)


@jax.jit
def kernel(u, i, emb_u, emb_i):
    return _sc_call(u.astype(jnp.int32), i.astype(jnp.int32), emb_u, emb_i)
